# baseline (device time: 59729 ns/iter reference)
import numpy as np

import jax
import jax.numpy as jnp
from jax import lax
from jax.experimental import pallas as pl
from jax.experimental.pallas import tpu as pltpu

N_DEV = 4
B, SQ_LOCAL, D_MODEL = 2, 256, 768
HQ, DH = 4, 64
D_QKV = HQ * DH
ROWS = B * SQ_LOCAL
SEQ = N_DEV * SQ_LOCAL


def _rope_tables():
    inv = 1.0 / (10000.0 ** (np.arange(0, DH, 2) / DH))
    pos = np.arange(SEQ)[:, None] * inv[None, :]
    cos = np.repeat(np.cos(pos), 2, axis=-1).astype(np.float32)
    sin = np.repeat(np.sin(pos), 2, axis=-1).astype(np.float32)
    cos_t = np.tile(cos, (1, HQ))
    sin_t = np.tile(sin, (1, HQ))
    P = np.kron(np.eye(D_QKV // 2), np.array([[0.0, 1.0], [-1.0, 0.0]]))
    return cos_t, sin_t, P.astype(np.float32)


def kernel(x, Wq, Wk, Wv, Wo):
    cos_t, sin_t, P = _rope_tables()

    def body(x_ref, wq_ref, wk_ref, wv_ref, wo_ref, cos_ref, sin_ref, p_ref,
             out_ref, kbuf, vbuf, ksend, krecv, vsend, vrecv):
        my_pos = lax.axis_index("i")
        left = (my_pos - 1) % N_DEV
        right = (my_pos + 1) % N_DEV

        barrier_sem = pltpu.get_barrier_semaphore()
        for nbr in (left, right):
            pl.semaphore_signal(
                barrier_sem, inc=1,
                device_id=(nbr,), device_id_type=pl.DeviceIdType.MESH,
            )
        pl.semaphore_wait(barrier_sem, 2)

        xf = jnp.concatenate([x_ref[0], x_ref[1]], axis=0)
        q = jnp.dot(xf, wq_ref[:], preferred_element_type=jnp.float32)
        k = jnp.dot(xf, wk_ref[:], preferred_element_type=jnp.float32)
        v = jnp.dot(xf, wv_ref[:], preferred_element_type=jnp.float32)

        cos_l = cos_ref[pl.ds(my_pos * SQ_LOCAL, SQ_LOCAL), :]
        sin_l = sin_ref[pl.ds(my_pos * SQ_LOCAL, SQ_LOCAL), :]
        cos2 = jnp.concatenate([cos_l, cos_l], axis=0)
        sin2 = jnp.concatenate([sin_l, sin_l], axis=0)
        pmat = p_ref[:]
        q = q * cos2 + jnp.dot(q, pmat, preferred_element_type=jnp.float32) * sin2
        k = k * cos2 + jnp.dot(k, pmat, preferred_element_type=jnp.float32) * sin2

        kbuf[0] = k
        vbuf[0] = v

        for h in range(N_DEV - 1):
            k_rdma = pltpu.make_async_remote_copy(
                src_ref=kbuf.at[h],
                dst_ref=kbuf.at[h + 1],
                send_sem=ksend.at[h],
                recv_sem=krecv.at[h],
                device_id=(right,),
                device_id_type=pl.DeviceIdType.MESH,
            )
            v_rdma = pltpu.make_async_remote_copy(
                src_ref=vbuf.at[h],
                dst_ref=vbuf.at[h + 1],
                send_sem=vsend.at[h],
                recv_sem=vrecv.at[h],
                device_id=(right,),
                device_id_type=pl.DeviceIdType.MESH,
            )
            k_rdma.start()
            v_rdma.start()
            k_rdma.wait()
            v_rdma.wait()

        ks = [kbuf[s] for s in range(N_DEV)]
        vs = [vbuf[s] for s in range(N_DEV)]

        batch_rows = []
        for b in range(B):
            row = slice(b * SQ_LOCAL, (b + 1) * SQ_LOCAL)
            head_cols = []
            for hq in range(HQ):
                col = slice(hq * DH, (hq + 1) * DH)
                qbh = q[row, col]
                s_parts = [
                    lax.dot_general(
                        qbh, ks[s][row, col],
                        dimension_numbers=(((1,), (1,)), ((), ())),
                        preferred_element_type=jnp.float32,
                    )
                    for s in range(N_DEV)
                ]
                scores = jnp.concatenate(s_parts, axis=1) * 0.125
                m = jnp.max(scores, axis=1, keepdims=True)
                e = jnp.exp(scores - m)
                w = e / jnp.sum(e, axis=1, keepdims=True)
                ctx = sum(
                    jnp.dot(
                        w[:, s * SQ_LOCAL:(s + 1) * SQ_LOCAL], vs[s][row, col],
                        preferred_element_type=jnp.float32,
                    )
                    for s in range(N_DEV)
                )
                head_cols.append(ctx)
            batch_rows.append(jnp.concatenate(head_cols, axis=1))
        ctx_all = jnp.concatenate(batch_rows, axis=0)

        out = jnp.dot(ctx_all, wo_ref[:], preferred_element_type=jnp.float32)
        out_ref[0] = out[:SQ_LOCAL]
        out_ref[1] = out[SQ_LOCAL:]

    return pl.pallas_call(
        body,
        out_shape=jax.ShapeDtypeStruct((B, SQ_LOCAL, D_MODEL), jnp.float32),
        in_specs=[pl.BlockSpec(memory_space=pltpu.VMEM)] * 8,
        out_specs=pl.BlockSpec(memory_space=pltpu.VMEM),
        scratch_shapes=[
            pltpu.VMEM((N_DEV, ROWS, D_QKV), jnp.float32),
            pltpu.VMEM((N_DEV, ROWS, D_QKV), jnp.float32),
            pltpu.SemaphoreType.DMA((N_DEV - 1,)),
            pltpu.SemaphoreType.DMA((N_DEV - 1,)),
            pltpu.SemaphoreType.DMA((N_DEV - 1,)),
            pltpu.SemaphoreType.DMA((N_DEV - 1,)),
        ],
        compiler_params=pltpu.CompilerParams(collective_id=0),
    )(x, Wq, Wk, Wv, Wo, jnp.asarray(cos_t), jnp.asarray(sin_t), jnp.asarray(P))


# device time: 36997 ns/iter; 1.6144x vs baseline; 1.6144x over previous
import numpy as np

import jax
import jax.numpy as jnp
from jax import lax
from jax.experimental import pallas as pl
from jax.experimental.pallas import tpu as pltpu

N_DEV = 4
B, SQ_LOCAL, D_MODEL = 2, 256, 768
HQ, DH = 4, 64
D_QKV = HQ * DH
ROWS = B * SQ_LOCAL

SLOT_OWN, SLOT_L, SLOT_R, SLOT_O = 0, 1, 2, 3


def _rope_tables():
    inv = 1.0 / (10000.0 ** (np.arange(0, DH, 2) / DH))
    pos = np.arange(N_DEV * SQ_LOCAL)[:, None] * inv[None, :]
    cos = np.repeat(np.cos(pos), 2, axis=-1).astype(np.float32)
    sin = np.repeat(np.sin(pos), 2, axis=-1).astype(np.float32)
    cos_t = np.tile(cos, (1, HQ))
    sin_t = np.tile(sin, (1, HQ))
    P = np.kron(np.eye(D_QKV // 2), np.array([[0.0, 1.0], [-1.0, 0.0]]))
    return cos_t, sin_t, P.astype(np.float32)


def kernel(x, Wq, Wk, Wv, Wo):
    cos_t, sin_t, P = _rope_tables()

    def body(x_ref, wq_ref, wk_ref, wv_ref, wo_ref, cos_ref, sin_ref, p_ref,
             out_ref, kbuf, vbuf, ksend, krecv, vsend, vrecv):
        my_pos = lax.axis_index("i")
        left = (my_pos - 1) % N_DEV
        right = (my_pos + 1) % N_DEV

        barrier_sem = pltpu.get_barrier_semaphore()
        for nbr in (left, right):
            pl.semaphore_signal(
                barrier_sem, inc=1,
                device_id=(nbr,), device_id_type=pl.DeviceIdType.MESH,
            )
        pl.semaphore_wait(barrier_sem, 2)

        def rdma(buf, send, recv, src, dst, sem, dev):
            return pltpu.make_async_remote_copy(
                src_ref=buf.at[src] if isinstance(src, int) else buf.at[src],
                dst_ref=buf.at[dst] if isinstance(dst, int) else buf.at[dst],
                send_sem=send.at[sem],
                recv_sem=recv.at[sem],
                device_id=(dev,),
                device_id_type=pl.DeviceIdType.MESH,
            )

        xf = jnp.concatenate([x_ref[0], x_ref[1]], axis=0)
        cos_l = cos_ref[pl.ds(my_pos * SQ_LOCAL, SQ_LOCAL), :]
        sin_l = sin_ref[pl.ds(my_pos * SQ_LOCAL, SQ_LOCAL), :]
        cos2 = jnp.concatenate([cos_l, cos_l], axis=0)
        sin2 = jnp.concatenate([sin_l, sin_l], axis=0)
        pmat = p_ref[:]

        def rope(t):
            return t * cos2 + jnp.dot(
                t, pmat, preferred_element_type=jnp.float32) * sin2

        k = rope(jnp.dot(xf, wk_ref[:], preferred_element_type=jnp.float32))
        kbuf[pl.ds(0, 2)] = k.reshape(B, SQ_LOCAL, D_QKV)
        k_cw1 = rdma(kbuf, ksend, krecv, pl.ds(0, 2), pl.ds(2, 2), 0, right)
        k_ccw1 = rdma(kbuf, ksend, krecv, pl.ds(0, 2), pl.ds(4, 2), 1, left)
        k_cw1.start()
        k_ccw1.start()

        v = jnp.dot(xf, wv_ref[:], preferred_element_type=jnp.float32)
        vbuf[pl.ds(0, 2)] = v.reshape(B, SQ_LOCAL, D_QKV)
        v_cw1 = rdma(vbuf, vsend, vrecv, pl.ds(0, 2), pl.ds(2, 2), 0, right)
        v_ccw1 = rdma(vbuf, vsend, vrecv, pl.ds(0, 2), pl.ds(4, 2), 1, left)
        v_cw1.start()
        v_ccw1.start()

        q = rope(jnp.dot(xf, wq_ref[:], preferred_element_type=jnp.float32))

        acc = [[None] * HQ for _ in range(B)]
        lse = [[None] * HQ for _ in range(B)]

        def process(slot, batches=(0, 1)):
            for b in batches:
                kv = kbuf[2 * slot + b]
                vv = vbuf[2 * slot + b]
                qb = q[b * SQ_LOCAL:(b + 1) * SQ_LOCAL, :]
                for h in range(HQ):
                    col = slice(h * DH, (h + 1) * DH)
                    sc = lax.dot_general(
                        qb[:, col], kv[:, col],
                        dimension_numbers=(((1,), (1,)), ((), ())),
                        preferred_element_type=jnp.float32,
                    ) * 0.125
                    e = jnp.exp(sc)
                    pv = jnp.dot(e, vv[:, col],
                                 preferred_element_type=jnp.float32)
                    ls = jnp.sum(e, axis=1, keepdims=True)
                    if acc[b][h] is None:
                        acc[b][h], lse[b][h] = pv, ls
                    else:
                        acc[b][h] = acc[b][h] + pv
                        lse[b][h] = lse[b][h] + ls

            return None

        process(SLOT_OWN)

        k_cw1.wait_recv()
        v_cw1.wait_recv()
        k_cw2 = rdma(kbuf, ksend, krecv, 2 * SLOT_L + 0, 2 * SLOT_O + 0, 2, right)
        v_cw2 = rdma(vbuf, vsend, vrecv, 2 * SLOT_L + 0, 2 * SLOT_O + 0, 2, right)
        k_cw2.start()
        v_cw2.start()
        process(SLOT_L)

        k_ccw1.wait_recv()
        v_ccw1.wait_recv()
        k_ccw2 = rdma(kbuf, ksend, krecv, 2 * SLOT_R + 1, 2 * SLOT_O + 1, 3, left)
        v_ccw2 = rdma(vbuf, vsend, vrecv, 2 * SLOT_R + 1, 2 * SLOT_O + 1, 3, left)
        k_ccw2.start()
        v_ccw2.start()
        process(SLOT_R)

        k_cw2.wait_recv()
        v_cw2.wait_recv()
        process(SLOT_O, batches=(0,))
        k_ccw2.wait_recv()
        v_ccw2.wait_recv()
        process(SLOT_O, batches=(1,))

        for b in range(B):
            ctx_b = jnp.concatenate(
                [acc[b][h] / lse[b][h] for h in range(HQ)], axis=1
            )
            out_ref[b] = jnp.dot(ctx_b, wo_ref[:],
                                 preferred_element_type=jnp.float32)

        for d in (k_cw1, k_ccw1, v_cw1, v_ccw1, k_cw2, v_cw2, k_ccw2, v_ccw2):
            d.wait_send()

    return pl.pallas_call(
        body,
        out_shape=jax.ShapeDtypeStruct((B, SQ_LOCAL, D_MODEL), jnp.float32),
        in_specs=[pl.BlockSpec(memory_space=pltpu.VMEM)] * 8,
        out_specs=pl.BlockSpec(memory_space=pltpu.VMEM),
        scratch_shapes=[
            pltpu.VMEM((2 * N_DEV, SQ_LOCAL, D_QKV), jnp.float32),
            pltpu.VMEM((2 * N_DEV, SQ_LOCAL, D_QKV), jnp.float32),
            pltpu.SemaphoreType.DMA((4,)),
            pltpu.SemaphoreType.DMA((4,)),
            pltpu.SemaphoreType.DMA((4,)),
            pltpu.SemaphoreType.DMA((4,)),
        ],
        compiler_params=pltpu.CompilerParams(collective_id=0),
    )(x, Wq, Wk, Wv, Wo, jnp.asarray(cos_t), jnp.asarray(sin_t), jnp.asarray(P))


# device time: 29068 ns/iter; 2.0548x vs baseline; 1.2728x over previous
import numpy as np

import jax
import jax.numpy as jnp
from jax import lax
from jax.experimental import pallas as pl
from jax.experimental.pallas import tpu as pltpu

N_DEV = 4
B, SQ_LOCAL, D_MODEL = 2, 256, 768
HQ, DH = 4, 64
D_QKV = HQ * DH
ROWS = B * SQ_LOCAL

SLOT_OWN, SLOT_L, SLOT_R, SLOT_O = 0, 1, 2, 3


def _rope_tables():
    inv = 1.0 / (10000.0 ** (np.arange(0, DH, 2) / DH))
    pos = np.arange(N_DEV * SQ_LOCAL)[:, None] * inv[None, :]
    cos = np.repeat(np.cos(pos), 2, axis=-1).astype(np.float32)
    sin = np.repeat(np.sin(pos), 2, axis=-1).astype(np.float32)
    cos_t = np.tile(cos, (1, HQ))
    sin_t = np.tile(sin, (1, HQ))
    P = np.kron(np.eye(D_QKV // 2), np.array([[0.0, 1.0], [-1.0, 0.0]]))
    return cos_t, sin_t, P.astype(np.float32)


def kernel(x, Wq, Wk, Wv, Wo):
    cos_t, sin_t, P = _rope_tables()

    def body(x_ref, wq_ref, wk_ref, wv_ref, wo_ref, cos_ref, sin_ref, p_ref,
             out_ref, kbuf, vbuf, ksend, krecv, vsend, vrecv):
        my_pos = lax.axis_index("i")
        left = (my_pos - 1) % N_DEV
        right = (my_pos + 1) % N_DEV

        barrier_sem = pltpu.get_barrier_semaphore()
        for nbr in (left, right):
            pl.semaphore_signal(
                barrier_sem, inc=1,
                device_id=(nbr,), device_id_type=pl.DeviceIdType.MESH,
            )
        pl.semaphore_wait(barrier_sem, 2)

        def rdma(buf, send, recv, src, dst, sem, dev):
            return pltpu.make_async_remote_copy(
                src_ref=buf.at[src],
                dst_ref=buf.at[dst],
                send_sem=send.at[sem],
                recv_sem=recv.at[sem],
                device_id=(dev,),
                device_id_type=pl.DeviceIdType.MESH,
            )

        bf16 = jnp.bfloat16
        xf = jnp.concatenate([x_ref[0], x_ref[1]], axis=0)
        xf_b = xf.astype(bf16)
        cos_l = cos_ref[pl.ds(my_pos * SQ_LOCAL, SQ_LOCAL), :]
        sin_l = sin_ref[pl.ds(my_pos * SQ_LOCAL, SQ_LOCAL), :]
        cos2 = jnp.concatenate([cos_l, cos_l], axis=0)
        sin2 = jnp.concatenate([sin_l, sin_l], axis=0)
        pmat = p_ref[:].astype(bf16)

        def rope(t):
            rot = jnp.dot(t.astype(bf16), pmat,
                          preferred_element_type=jnp.float32)
            return t * cos2 + rot * sin2

        k = rope(jnp.dot(xf_b, wk_ref[:].astype(bf16),
                         preferred_element_type=jnp.float32))
        kbuf[pl.ds(0, 2)] = k.astype(bf16).reshape(B, SQ_LOCAL, D_QKV)
        k_cw1 = rdma(kbuf, ksend, krecv, pl.ds(0, 2), pl.ds(2, 2), 0, right)
        k_ccw1 = rdma(kbuf, ksend, krecv, pl.ds(0, 2), pl.ds(4, 2), 1, left)
        k_cw1.start()
        k_ccw1.start()

        v = jnp.dot(xf_b, wv_ref[:].astype(bf16),
                    preferred_element_type=jnp.float32)
        vbuf[pl.ds(0, 2)] = v.astype(bf16).reshape(B, SQ_LOCAL, D_QKV)
        v_cw1 = rdma(vbuf, vsend, vrecv, pl.ds(0, 2), pl.ds(2, 2), 0, right)
        v_ccw1 = rdma(vbuf, vsend, vrecv, pl.ds(0, 2), pl.ds(4, 2), 1, left)
        v_cw1.start()
        v_ccw1.start()

        q = rope(jnp.dot(xf_b, wq_ref[:].astype(bf16),
                         preferred_element_type=jnp.float32)).astype(bf16)
        wo_b = wo_ref[:].astype(bf16)

        acc = [[None] * HQ for _ in range(B)]
        lse = [[None] * HQ for _ in range(B)]

        def process(slot, batches=(0, 1)):
            for b in batches:
                kv = kbuf[2 * slot + b]
                vv = vbuf[2 * slot + b]
                qb = q[b * SQ_LOCAL:(b + 1) * SQ_LOCAL, :]
                for h in range(HQ):
                    col = slice(h * DH, (h + 1) * DH)
                    sc = lax.dot_general(
                        qb[:, col], kv[:, col],
                        dimension_numbers=(((1,), (1,)), ((), ())),
                        preferred_element_type=jnp.float32,
                    ) * 0.125
                    e = jnp.exp(sc)
                    pv = jnp.dot(e.astype(bf16), vv[:, col],
                                 preferred_element_type=jnp.float32)
                    ls = jnp.sum(e, axis=1, keepdims=True)
                    if acc[b][h] is None:
                        acc[b][h], lse[b][h] = pv, ls
                    else:
                        acc[b][h] = acc[b][h] + pv
                        lse[b][h] = lse[b][h] + ls

        def emit(b):
            ctx_b = jnp.concatenate(
                [acc[b][h] / lse[b][h] for h in range(HQ)], axis=1
            )
            out_ref[b] = jnp.dot(ctx_b.astype(bf16), wo_b,
                                 preferred_element_type=jnp.float32)

        process(SLOT_OWN)

        k_cw1.wait_recv()
        v_cw1.wait_recv()
        k_cw2 = rdma(kbuf, ksend, krecv, 2 * SLOT_L + 0, 2 * SLOT_O + 0, 2, right)
        v_cw2 = rdma(vbuf, vsend, vrecv, 2 * SLOT_L + 0, 2 * SLOT_O + 0, 2, right)
        k_cw2.start()
        v_cw2.start()
        process(SLOT_L)

        k_ccw1.wait_recv()
        v_ccw1.wait_recv()
        k_ccw2 = rdma(kbuf, ksend, krecv, 2 * SLOT_R + 1, 2 * SLOT_O + 1, 3, left)
        v_ccw2 = rdma(vbuf, vsend, vrecv, 2 * SLOT_R + 1, 2 * SLOT_O + 1, 3, left)
        k_ccw2.start()
        v_ccw2.start()
        process(SLOT_R)

        k_cw2.wait_recv()
        v_cw2.wait_recv()
        process(SLOT_O, batches=(0,))
        emit(0)

        k_ccw2.wait_recv()
        v_ccw2.wait_recv()
        process(SLOT_O, batches=(1,))
        emit(1)

        for d in (k_cw1, k_ccw1, v_cw1, v_ccw1, k_cw2, v_cw2, k_ccw2, v_ccw2):
            d.wait_send()

    return pl.pallas_call(
        body,
        out_shape=jax.ShapeDtypeStruct((B, SQ_LOCAL, D_MODEL), jnp.float32),
        in_specs=[pl.BlockSpec(memory_space=pltpu.VMEM)] * 8,
        out_specs=pl.BlockSpec(memory_space=pltpu.VMEM),
        scratch_shapes=[
            pltpu.VMEM((2 * N_DEV, SQ_LOCAL, D_QKV), jnp.bfloat16),
            pltpu.VMEM((2 * N_DEV, SQ_LOCAL, D_QKV), jnp.bfloat16),
            pltpu.SemaphoreType.DMA((4,)),
            pltpu.SemaphoreType.DMA((4,)),
            pltpu.SemaphoreType.DMA((4,)),
            pltpu.SemaphoreType.DMA((4,)),
        ],
        compiler_params=pltpu.CompilerParams(collective_id=0),
    )(x, Wq, Wk, Wv, Wo, jnp.asarray(cos_t), jnp.asarray(sin_t), jnp.asarray(P))


# device time: 28235 ns/iter; 2.1154x vs baseline; 1.0295x over previous
import numpy as np

import jax
import jax.numpy as jnp
from jax import lax
from jax.experimental import pallas as pl
from jax.experimental.pallas import tpu as pltpu

N_DEV = 4
B, SQ_LOCAL, D_MODEL = 2, 256, 768
HQ, DH = 4, 64
D_QKV = HQ * DH
ROWS = B * SQ_LOCAL

SLOT_OWN, SLOT_L, SLOT_R, SLOT_O = 0, 1, 2, 3


def _rope_tables():
    inv = 1.0 / (10000.0 ** (np.arange(0, DH, 2) / DH))
    pos = np.arange(N_DEV * SQ_LOCAL)[:, None] * inv[None, :]
    cos = np.repeat(np.cos(pos), 2, axis=-1).astype(np.float32)
    sin = np.repeat(np.sin(pos), 2, axis=-1).astype(np.float32)
    cos_t = np.tile(cos, (1, HQ))
    sin_t = np.tile(sin, (1, HQ))
    P = np.kron(np.eye(D_QKV // 2), np.array([[0.0, 1.0], [-1.0, 0.0]]))
    return cos_t, sin_t, P.astype(np.float32)


def kernel(x, Wq, Wk, Wv, Wo):
    cos_t, sin_t, P = _rope_tables()

    def body(x_ref, wq_ref, wk_ref, wv_ref, wo_ref, cos_ref, sin_ref, p_ref,
             out_ref, kbuf, vbuf, ksend, krecv, vsend, vrecv):
        my_pos = lax.axis_index("i")
        left = (my_pos - 1) % N_DEV
        right = (my_pos + 1) % N_DEV

        barrier_sem = pltpu.get_barrier_semaphore()
        for nbr in (left, right):
            pl.semaphore_signal(
                barrier_sem, inc=1,
                device_id=(nbr,), device_id_type=pl.DeviceIdType.MESH,
            )
        pl.semaphore_wait(barrier_sem, 2)

        def rdma(buf, send, recv, src, dst, sem, dev):
            return pltpu.make_async_remote_copy(
                src_ref=buf.at[src],
                dst_ref=buf.at[dst],
                send_sem=send.at[sem],
                recv_sem=recv.at[sem],
                device_id=(dev,),
                device_id_type=pl.DeviceIdType.MESH,
            )

        bf16 = jnp.bfloat16
        xf = jnp.concatenate([x_ref[0], x_ref[1]], axis=0)
        xf_b = xf.astype(bf16)
        cos_l = cos_ref[pl.ds(my_pos * SQ_LOCAL, SQ_LOCAL), :]
        sin_l = sin_ref[pl.ds(my_pos * SQ_LOCAL, SQ_LOCAL), :]
        cos2 = jnp.concatenate([cos_l, cos_l], axis=0)
        sin2 = jnp.concatenate([sin_l, sin_l], axis=0)
        pmat = p_ref[:].astype(bf16)

        def rope(t):
            rot = jnp.dot(t.astype(bf16), pmat,
                          preferred_element_type=jnp.float32)
            return t * cos2 + rot * sin2

        k = rope(jnp.dot(xf_b, wk_ref[:].astype(bf16),
                         preferred_element_type=jnp.float32))
        kbuf[pl.ds(0, 2)] = k.astype(bf16).reshape(B, SQ_LOCAL, D_QKV)
        k_cw1 = rdma(kbuf, ksend, krecv, pl.ds(0, 2), pl.ds(2, 2), 0, right)
        k_ccw1 = rdma(kbuf, ksend, krecv, pl.ds(0, 2), pl.ds(4, 2), 1, left)
        k_cw1.start()
        k_ccw1.start()

        v = jnp.dot(xf_b, wv_ref[:].astype(bf16),
                    preferred_element_type=jnp.float32)
        vbuf[pl.ds(0, 2)] = v.astype(bf16).reshape(B, SQ_LOCAL, D_QKV)
        v_cw1 = rdma(vbuf, vsend, vrecv, pl.ds(0, 2), pl.ds(2, 2), 0, right)
        v_ccw1 = rdma(vbuf, vsend, vrecv, pl.ds(0, 2), pl.ds(4, 2), 1, left)
        v_cw1.start()
        v_ccw1.start()

        q = (rope(jnp.dot(xf_b, wq_ref[:].astype(bf16),
                          preferred_element_type=jnp.float32))
             * 0.125).astype(bf16)
        wo_b = wo_ref[:].astype(bf16)

        acc = [[None] * HQ for _ in range(B)]
        lse = [[None] * HQ for _ in range(B)]

        def process(slot, batches=(0, 1)):
            for b in batches:
                kv = kbuf[2 * slot + b]
                vv = vbuf[2 * slot + b]
                qb = q[b * SQ_LOCAL:(b + 1) * SQ_LOCAL, :]
                for h in range(HQ):
                    col = slice(h * DH, (h + 1) * DH)
                    sc = lax.dot_general(
                        qb[:, col], kv[:, col],
                        dimension_numbers=(((1,), (1,)), ((), ())),
                        preferred_element_type=jnp.float32,
                    )
                    e = jnp.exp(sc)
                    pv = jnp.dot(e.astype(bf16), vv[:, col],
                                 preferred_element_type=jnp.float32)
                    ls = jnp.sum(e, axis=1, keepdims=True)
                    if acc[b][h] is None:
                        acc[b][h], lse[b][h] = pv, ls
                    else:
                        acc[b][h] = acc[b][h] + pv
                        lse[b][h] = lse[b][h] + ls

        def emit(b):
            ctx_b = jnp.concatenate(
                [acc[b][h] / lse[b][h] for h in range(HQ)], axis=1
            )
            out_ref[b] = jnp.dot(ctx_b.astype(bf16), wo_b,
                                 preferred_element_type=jnp.float32)

        process(SLOT_OWN)

        k_cw1.wait_recv()
        v_cw1.wait_recv()
        k_cw2 = rdma(kbuf, ksend, krecv, 2 * SLOT_L + 0, 2 * SLOT_O + 0, 2, right)
        v_cw2 = rdma(vbuf, vsend, vrecv, 2 * SLOT_L + 0, 2 * SLOT_O + 0, 2, right)
        k_cw2.start()
        v_cw2.start()
        k_ccw1.wait_recv()
        v_ccw1.wait_recv()
        k_ccw2 = rdma(kbuf, ksend, krecv, 2 * SLOT_R + 1, 2 * SLOT_O + 1, 3, left)
        v_ccw2 = rdma(vbuf, vsend, vrecv, 2 * SLOT_R + 1, 2 * SLOT_O + 1, 3, left)
        k_ccw2.start()
        v_ccw2.start()

        process(SLOT_L)
        process(SLOT_R)

        k_cw2.wait_recv()
        v_cw2.wait_recv()
        process(SLOT_O, batches=(0,))
        emit(0)

        k_ccw2.wait_recv()
        v_ccw2.wait_recv()
        process(SLOT_O, batches=(1,))
        emit(1)

        for d in (k_cw1, k_ccw1, v_cw1, v_ccw1, k_cw2, v_cw2, k_ccw2, v_ccw2):
            d.wait_send()

    return pl.pallas_call(
        body,
        out_shape=jax.ShapeDtypeStruct((B, SQ_LOCAL, D_MODEL), jnp.float32),
        in_specs=[pl.BlockSpec(memory_space=pltpu.VMEM)] * 8,
        out_specs=pl.BlockSpec(memory_space=pltpu.VMEM),
        scratch_shapes=[
            pltpu.VMEM((2 * N_DEV, SQ_LOCAL, D_QKV), jnp.bfloat16),
            pltpu.VMEM((2 * N_DEV, SQ_LOCAL, D_QKV), jnp.bfloat16),
            pltpu.SemaphoreType.DMA((4,)),
            pltpu.SemaphoreType.DMA((4,)),
            pltpu.SemaphoreType.DMA((4,)),
            pltpu.SemaphoreType.DMA((4,)),
        ],
        compiler_params=pltpu.CompilerParams(collective_id=0),
    )(x, Wq, Wk, Wv, Wo, jnp.asarray(cos_t), jnp.asarray(sin_t), jnp.asarray(P))
